# Initial kernel scaffold; baseline (speedup 1.0000x reference)
import jax
import jax.numpy as jnp
from jax.experimental import pallas as pl


def kernel(x, edge_index, W1, b1, Wg0, bg0, Wp0, bp0, Wa0, ba0, Wg1, bg1, Wp1, bp1, Wa1, ba1, W2, b2):
    raise NotImplementedError("write your pallas kernel here")


# speedup vs baseline: 16.2853x; 16.2853x over previous
import functools

import jax
import jax.numpy as jnp
from jax import lax
from jax.experimental import pallas as pl
from jax.experimental.pallas import tpu as pltpu
from jax.experimental.pallas import tpu_sc as plsc

N = 10000
E = 320000
H = 128
OUT = 64

NC = 2
NS = 16
NW = NC * NS
CHUNK = 128
NCHUNK = 81
PER_W = CHUNK * NCHUNK
E_PAD = PER_W * NW
NROW = 10240
ROWS_PER_TILE = NROW // NS
DUMMY_DST = NROW - 1


@functools.cache
def _make_deg_kernel():
    mesh = plsc.VectorSubcoreMesh(core_axis_name="c", subcore_axis_name="s")
    return pl.kernel(
        _deg_body,
        out_type=jax.ShapeDtypeStruct((NC, NROW, 16), jnp.float32),
        mesh=mesh,
        scratch_types=[
            pltpu.VMEM((NCHUNK, CHUNK), jnp.int32),
            pltpu.VMEM((CHUNK, 16), jnp.float32),
            pltpu.VMEM((16, 16), jnp.float32),
            pltpu.VMEM_SHARED((NROW, 16), jnp.float32),
        ],
    )


def _deg_body(dst_hbm, out_hbm, dst_v, ones_v, zeros_v, acc):
    cid = lax.axis_index("c")
    sid = lax.axis_index("s")
    wid = cid * NS + sid
    for r in range(16):
        zeros_v[r, :] = jnp.zeros((16,), jnp.float32)
    for r in range(CHUNK):
        ones_v[r, :] = jnp.ones((16,), jnp.float32)

    def zero_body(i, carry):
        pltpu.sync_copy(zeros_v, acc.at[pl.ds(sid * ROWS_PER_TILE + i * 16, 16)])
        return carry
    lax.fori_loop(0, ROWS_PER_TILE // 16, zero_body, 0)
    plsc.subcore_barrier()

    pltpu.sync_copy(dst_hbm.at[wid], dst_v)

    def body(j, carry):
        pltpu.sync_copy(ones_v, acc.at[dst_v.at[j]], add=True)
        return carry
    lax.fori_loop(0, NCHUNK, body, 0)
    plsc.subcore_barrier()

    sl = pl.ds(sid * ROWS_PER_TILE, ROWS_PER_TILE)
    pltpu.sync_copy(acc.at[sl], out_hbm.at[cid, sl])


@functools.cache
def _make_agg_kernel():
    mesh = plsc.VectorSubcoreMesh(core_axis_name="c", subcore_axis_name="s")
    return pl.kernel(
        _agg_body,
        out_type=jax.ShapeDtypeStruct((NC, NROW, H), jnp.float32),
        mesh=mesh,
        scratch_types=[
            pltpu.VMEM((NCHUNK, CHUNK), jnp.int32),
            pltpu.VMEM((NCHUNK, CHUNK), jnp.int32),
            pltpu.VMEM((CHUNK, H), jnp.float32),
            pltpu.VMEM((16, H), jnp.float32),
            pltpu.VMEM_SHARED((NROW, H), jnp.float32),
            pltpu.SemaphoreType.DMA,
        ],
    )


def _agg_body(g_hbm, src_hbm, dst_hbm, out_hbm,
              src_v, dst_v, rows_v, zeros_v, acc, sem):
    cid = lax.axis_index("c")
    sid = lax.axis_index("s")
    wid = cid * NS + sid
    for r in range(16):
        for c in range(H // 16):
            zeros_v[r, pl.ds(c * 16, 16)] = jnp.zeros((16,), jnp.float32)

    def zero_body(i, carry):
        pltpu.sync_copy(zeros_v, acc.at[pl.ds(sid * ROWS_PER_TILE + i * 16, 16)])
        return carry
    lax.fori_loop(0, ROWS_PER_TILE // 16, zero_body, 0)
    plsc.subcore_barrier()

    pltpu.sync_copy(src_hbm.at[wid], src_v)
    pltpu.sync_copy(dst_hbm.at[wid], dst_v)

    def body(j, carry):
        pltpu.async_copy(g_hbm.at[src_v.at[j]], rows_v, sem).wait()
        pltpu.sync_copy(rows_v, acc.at[dst_v.at[j]], add=True)
        return carry
    lax.fori_loop(0, NCHUNK, body, 0)
    plsc.subcore_barrier()

    sl = pl.ds(sid * ROWS_PER_TILE, ROWS_PER_TILE)
    pltpu.sync_copy(acc.at[sl], out_hbm.at[cid, sl])


def _dot_t(a, b):
    return lax.dot_general(a, b, (((1,), (1,)), ((), ())),
                           preferred_element_type=jnp.float32)


def _dis(d0_ref, d1_ref):
    return lax.rsqrt(d0_ref[...] + d1_ref[...])


def _a_body(x_ref, w1_ref, b1_ref, wg_ref, d0_ref, d1_ref, g_ref):
    dis = _dis(d0_ref, d1_ref)
    x1 = jnp.maximum(_dot_t(x_ref[...], w1_ref[...]) + b1_ref[...], 0.0)
    g_ref[...] = _dot_t(x1, wg_ref[...]) * dis


def _attn_x(p0_ref, p1_ref, d0_ref, d1_ref, bg_ref, wp_ref, bp_ref,
            wa_ref, ba_ref):
    dis = _dis(d0_ref, d1_ref)
    h = jnp.maximum((p0_ref[...] + p1_ref[...]) * dis + bg_ref[...], 0.0)
    t = jnp.tanh(_dot_t(h, wp_ref[...]) + bp_ref[...])
    s = jnp.sum(t * wa_ref[...], axis=1, keepdims=True) + ba_ref[0, 0]
    m = jnp.max(s)
    e = jnp.exp(s - m)
    return h * (e / jnp.sum(e)), dis


def _d_body(p0_ref, p1_ref, d0_ref, d1_ref, bg_ref, wp_ref, bp_ref,
            wa_ref, ba_ref, wg_ref, g_ref):
    x2, dis = _attn_x(p0_ref, p1_ref, d0_ref, d1_ref, bg_ref, wp_ref,
                      bp_ref, wa_ref, ba_ref)
    g_ref[...] = _dot_t(x2, wg_ref[...]) * dis


def _f_body(p0_ref, p1_ref, d0_ref, d1_ref, bg_ref, wp_ref, bp_ref,
            wa_ref, ba_ref, w2_ref, b2_ref, o_ref):
    x3, _ = _attn_x(p0_ref, p1_ref, d0_ref, d1_ref, bg_ref, wp_ref,
                    bp_ref, wa_ref, ba_ref)
    o = _dot_t(x3, w2_ref[...]) + b2_ref[...]
    mr = jnp.max(o, axis=1, keepdims=True)
    lse = mr + jnp.log(jnp.sum(jnp.exp(o - mr), axis=1, keepdims=True))
    o_ref[...] = o - lse


_a_call = pl.pallas_call(_a_body, out_shape=jax.ShapeDtypeStruct((N, H), jnp.float32))
_d_call = pl.pallas_call(_d_body, out_shape=jax.ShapeDtypeStruct((N, H), jnp.float32))
_f_call = pl.pallas_call(_f_body, out_shape=jax.ShapeDtypeStruct((N, OUT), jnp.float32))


def kernel(x, edge_index, W1, b1, Wg0, bg0, Wp0, bp0, Wa0, ba0,
           Wg1, bg1, Wp1, bp1, Wa1, ba1, W2, b2):
    loop = jnp.arange(N, dtype=jnp.int32)
    npad = E_PAD - (E + N)
    src = jnp.concatenate([edge_index[0], loop,
                           jnp.zeros((npad,), jnp.int32)])
    dst = jnp.concatenate([edge_index[1], loop,
                           jnp.full((npad,), DUMMY_DST, jnp.int32)])
    src3 = src.reshape(NW, NCHUNK, CHUNK)
    dst3 = dst.reshape(NW, NCHUNK, CHUNK)

    degP = _make_deg_kernel()(dst3)
    d0 = degP[0, :N, 0:1]
    d1 = degP[1, :N, 0:1]

    b1r = b1.reshape(1, H)
    bg0r = bg0.reshape(1, H)
    bp0r = bp0.reshape(1, H)
    ba0r = ba0.reshape(1, 1)
    bg1r = bg1.reshape(1, H)
    bp1r = bp1.reshape(1, H)
    ba1r = ba1.reshape(1, 1)
    b2r = b2.reshape(1, OUT)

    g0 = _a_call(x, W1, b1r, Wg0, d0, d1)

    aggP0 = _make_agg_kernel()(g0, src3, dst3)
    g1 = _d_call(aggP0[0, :N, :], aggP0[1, :N, :], d0, d1,
                 bg0r, Wp0, bp0r, Wa0, ba0r, Wg1)

    aggP1 = _make_agg_kernel()(g1, src3, dst3)
    out = _f_call(aggP1[0, :N, :], aggP1[1, :N, :], d0, d1,
                  bg1r, Wp1, bp1r, Wa1, ba1r, W2, b2r)
    return out
